# baseline (speedup 1.0000x reference)
import functools

import jax
import jax.numpy as jnp
from jax import lax
from jax.experimental import pallas as pl
from jax.experimental.pallas import tpu as pltpu
from jax.experimental.pallas import tpu_sc as plsc

EMBED_DIM = 32
BATCH = 16384

_NC = 2
_NS = 16
_NW = _NC * _NS
_BPW = BATCH // _NW
_ICH = 128
_NCH = _BPW // _ICH


def _tt_body(uid_hbm, mid_hbm, ut_hbm, it_hbm, out_hbm,
             idx_u, idx_m, rows_u, rows_m, dots, tps, sem):
    wid = lax.axis_index("s") * _NC + lax.axis_index("c")
    base = wid * _BPW

    for j in range(_NCH):
        pltpu.sync_copy(uid_hbm.at[pl.ds(base + j * _ICH, _ICH)], idx_u.at[j])
        pltpu.sync_copy(mid_hbm.at[pl.ds(base + j * _ICH, _ICH)], idx_m.at[j])

    copies = []
    for j in range(_NCH):
        copies.append(pltpu.async_copy(
            ut_hbm.at[idx_u.at[j]], rows_u.at[pl.ds(j * _ICH, _ICH)], sem))
        copies.append(pltpu.async_copy(
            it_hbm.at[idx_m.at[j]], rows_m.at[pl.ds(j * _ICH, _ICH)], sem))
    for c in copies:
        c.wait()

    lane = lax.iota(jnp.int32, 16)

    def group(g, carry):
        for jj in range(16):
            r = g * 16 + jj
            s = (rows_u[r, 0:16] * rows_m[r, 0:16]
                 + rows_u[r, 16:32] * rows_m[r, 16:32])
            plsc.store_scatter(tps, [lane * 17 + jj], s)
        acc = tps[pl.ds(0, 16)]
        for i in range(1, 16):
            acc = acc + tps[pl.ds(i * 17, 16)]
        dots[pl.ds(g * 16, 16)] = 1.0 / (1.0 + jnp.exp(-acc))
        return carry

    lax.fori_loop(0, _BPW // 16, group, 0)
    pltpu.sync_copy(dots, out_hbm.at[pl.ds(base, _BPW)])


@jax.jit
def _two_tower(user_id, movie_id, user_table, item_table):
    mesh = plsc.VectorSubcoreMesh(core_axis_name="c", subcore_axis_name="s")
    return pl.kernel(
        _tt_body,
        out_type=jax.ShapeDtypeStruct((BATCH,), jnp.float32),
        mesh=mesh,
        compiler_params=pltpu.CompilerParams(
            needs_layout_passes=False, use_tc_tiling_on_sc=False),
        scratch_types=[
            pltpu.VMEM((_NCH, _ICH), jnp.int32),
            pltpu.VMEM((_NCH, _ICH), jnp.int32),
            pltpu.VMEM((_BPW, EMBED_DIM), jnp.float32),
            pltpu.VMEM((_BPW, EMBED_DIM), jnp.float32),
            pltpu.VMEM((_BPW,), jnp.float32),
            pltpu.VMEM((16 * 17, ), jnp.float32),
            pltpu.SemaphoreType.DMA,
        ],
    )(user_id, movie_id, user_table, item_table)


def kernel(user_id, movie_id, user_table, item_table):
    return _two_tower(user_id.astype(jnp.int32), movie_id.astype(jnp.int32),
                      user_table, item_table)
